# baseline (speedup 1.0000x reference)
import functools

import jax
import jax.numpy as jnp
from jax import lax
from jax.experimental import pallas as pl
from jax.experimental.pallas import tpu as pltpu
from jax.experimental.pallas import tpu_sc as plsc

B = 16384
ITEM_D = 32
CAT_D = 6
BRAND_D = 10
TAIL_D = CAT_D + BRAND_D
OUT_D = ITEM_D + TAIL_D

NC = 2
NS = 16
NW = NC * NS
BPW = B // NW
CHUNK = 128
NCHUNK = BPW // CHUNK
NGROUP = BPW // 16


def _body(item_idx_hbm, cat_idx_hbm, brand_idx_hbm,
          item_tbl_hbm, cat_tbl_hbm, brand_tbl_hbm,
          out_hbm,
          ii_v, ic_v, ib_v, ig, tb, cat_vm, brand_vm, sem):
    wid = lax.axis_index("s") * NC + lax.axis_index("c")
    base = wid * NCHUNK

    pltpu.sync_copy(item_idx_hbm.at[pl.ds(base, NCHUNK)], ii_v)
    pltpu.sync_copy(cat_idx_hbm.at[pl.ds(base, NCHUNK)], ic_v)
    pltpu.sync_copy(brand_idx_hbm.at[pl.ds(base, NCHUNK)], ib_v)
    pltpu.sync_copy(cat_tbl_hbm, cat_vm)
    pltpu.sync_copy(brand_tbl_hbm, brand_vm)

    for j in range(NCHUNK):
        for i in range(CHUNK // 16):
            sl = pl.ds(i * 16, 16)
            ii_v[j, sl] = ii_v[j, sl] + 1

    copies = [
        pltpu.make_async_copy(
            item_tbl_hbm.at[ii_v.at[j]],
            ig.at[pl.ds(j * CHUNK, CHUNK)], sem)
        for j in range(NCHUNK)
    ]
    for c in copies:
        c.start()

    iota = lax.iota(jnp.int32, 16)
    one = jnp.full((16,), 1, jnp.int32)
    for g in range(NGROUP):
        j, i = divmod(g, CHUNK // 16)
        sl = pl.ds(i * 16, 16)
        rows = iota + (g * 16)
        icp = ic_v[j, sl] + one
        ibp = ib_v[j, sl] + one
        for c in range(TAIL_D):
            col = jnp.full((16,), c, jnp.int32)
            if c < CAT_D:
                v = plsc.load_gather(
                    cat_vm, [icp, jnp.full((16,), c, jnp.int32)])
            else:
                v = plsc.load_gather(
                    brand_vm, [ibp, jnp.full((16,), c - CAT_D, jnp.int32)])
            plsc.store_scatter(tb, [rows, col], v)

    for c in copies:
        c.wait()

    rows = pl.ds(wid * BPW, BPW)
    pltpu.sync_copy(ig, out_hbm.at[rows, pl.ds(0, ITEM_D)])
    pltpu.sync_copy(tb, out_hbm.at[rows, pl.ds(ITEM_D, TAIL_D)])


def kernel(item_id, category_id, brand_id, item_table, cat_table, brand_table):
    mesh = plsc.VectorSubcoreMesh(core_axis_name="c", subcore_axis_name="s")
    k = functools.partial(
        pl.kernel,
        mesh=mesh,
        compiler_params=pltpu.CompilerParams(use_tc_tiling_on_sc=False,
                                             needs_layout_passes=False,
                                             disable_bounds_checks=True),
        out_type=jax.ShapeDtypeStruct((B, OUT_D), jnp.float32),
        scratch_types=[
            pltpu.VMEM((NCHUNK, CHUNK), jnp.int32),
            pltpu.VMEM((NCHUNK, CHUNK), jnp.int32),
            pltpu.VMEM((NCHUNK, CHUNK), jnp.int32),
            pltpu.VMEM((BPW, ITEM_D), jnp.float32),
            pltpu.VMEM((BPW, TAIL_D), jnp.float32),
            pltpu.VMEM((101, CAT_D), jnp.float32),
            pltpu.VMEM((201, BRAND_D), jnp.float32),
            pltpu.SemaphoreType.DMA,
        ],
    )(_body)
    return k(item_id.reshape(B // CHUNK, CHUNK),
             category_id.reshape(B // CHUNK, CHUNK),
             brand_id.reshape(B // CHUNK, CHUNK),
             item_table, cat_table, brand_table)


# speedup vs baseline: 1.0543x; 1.0543x over previous
import functools

import jax
import jax.numpy as jnp
from jax import lax
from jax.experimental import pallas as pl
from jax.experimental.pallas import tpu as pltpu
from jax.experimental.pallas import tpu_sc as plsc

B = 16384
ITEM_D = 32
CAT_D = 6
BRAND_D = 10
TAIL_D = CAT_D + BRAND_D
OUT_D = ITEM_D + TAIL_D

NC = 2
NS = 16
NW = NC * NS
BPW = B // NW
CHUNK = 128
NCHUNK = BPW // CHUNK
NGROUP = BPW // 16


def _body(item_idx_hbm, cat_idx_hbm, brand_idx_hbm,
          item_tbl_hbm, cat_tbl_hbm, brand_tbl_hbm,
          out_hbm,
          ii_v, ic_v, ib_v, ig, tb, cat_vm, brand_vm, sem, gsem):
    wid = lax.axis_index("s") * NC + lax.axis_index("c")
    base = wid * NCHUNK

    stage = [
        pltpu.make_async_copy(item_idx_hbm.at[pl.ds(base, NCHUNK)], ii_v, sem),
        pltpu.make_async_copy(cat_idx_hbm.at[pl.ds(base, NCHUNK)], ic_v, sem),
        pltpu.make_async_copy(brand_idx_hbm.at[pl.ds(base, NCHUNK)], ib_v, sem),
        pltpu.make_async_copy(cat_tbl_hbm, cat_vm, sem),
        pltpu.make_async_copy(brand_tbl_hbm, brand_vm, sem),
    ]
    for c in stage:
        c.start()
    for c in stage:
        c.wait()

    for j in range(NCHUNK):
        for i in range(CHUNK // 16):
            sl = pl.ds(i * 16, 16)
            ii_v[j, sl] = ii_v[j, sl] + 1

    copies = [
        pltpu.make_async_copy(
            item_tbl_hbm.at[ii_v.at[j]],
            ig.at[pl.ds(j * CHUNK, CHUNK)], gsem)
        for j in range(NCHUNK)
    ]
    for c in copies:
        c.start()

    iota = lax.iota(jnp.int32, 16)
    one = jnp.full((16,), 1, jnp.int32)
    cat_cols = [jnp.full((16,), c, jnp.int32) for c in range(CAT_D)]
    brand_cols = [jnp.full((16,), c, jnp.int32) for c in range(BRAND_D)]
    out_cols = [jnp.full((16,), c, jnp.int32) for c in range(TAIL_D)]
    for g in range(NGROUP):
        j, i = divmod(g, CHUNK // 16)
        sl = pl.ds(i * 16, 16)
        rows = iota + (g * 16)
        icp = ic_v[j, sl] + one
        ibp = ib_v[j, sl] + one
        for c in range(TAIL_D):
            if c < CAT_D:
                v = plsc.load_gather(cat_vm, [icp, cat_cols[c]])
            else:
                v = plsc.load_gather(brand_vm, [ibp, brand_cols[c - CAT_D]])
            plsc.store_scatter(tb, [rows, out_cols[c]], v)

    for c in copies:
        c.wait()

    rows = pl.ds(wid * BPW, BPW)
    pltpu.sync_copy(ig, out_hbm.at[rows, pl.ds(0, ITEM_D)])
    pltpu.sync_copy(tb, out_hbm.at[rows, pl.ds(ITEM_D, TAIL_D)])


def kernel(item_id, category_id, brand_id, item_table, cat_table, brand_table):
    mesh = plsc.VectorSubcoreMesh(core_axis_name="c", subcore_axis_name="s")
    k = functools.partial(
        pl.kernel,
        mesh=mesh,
        compiler_params=pltpu.CompilerParams(use_tc_tiling_on_sc=False,
                                             needs_layout_passes=False,
                                             disable_bounds_checks=True,
                                             skip_device_barrier=True),
        out_type=jax.ShapeDtypeStruct((B, OUT_D), jnp.float32),
        scratch_types=[
            pltpu.VMEM((NCHUNK, CHUNK), jnp.int32),
            pltpu.VMEM((NCHUNK, CHUNK), jnp.int32),
            pltpu.VMEM((NCHUNK, CHUNK), jnp.int32),
            pltpu.VMEM((BPW, ITEM_D), jnp.float32),
            pltpu.VMEM((BPW, TAIL_D), jnp.float32),
            pltpu.VMEM((101, CAT_D), jnp.float32),
            pltpu.VMEM((201, BRAND_D), jnp.float32),
            pltpu.SemaphoreType.DMA,
            pltpu.SemaphoreType.DMA,
        ],
    )(_body)
    return k(item_id.reshape(B // CHUNK, CHUNK),
             category_id.reshape(B // CHUNK, CHUNK),
             brand_id.reshape(B // CHUNK, CHUNK),
             item_table, cat_table, brand_table)


# speedup vs baseline: 1.0893x; 1.0332x over previous
import functools

import jax
import jax.numpy as jnp
from jax import lax
from jax.experimental import pallas as pl
from jax.experimental.pallas import tpu as pltpu
from jax.experimental.pallas import tpu_sc as plsc

B = 16384
ITEM_D = 32
CAT_D = 6
BRAND_D = 10
TAIL_D = CAT_D + BRAND_D
OUT_D = ITEM_D + TAIL_D

CAT_WORDS = 101 * CAT_D
BRAND_WORDS = 201 * BRAND_D

NC = 2
NS = 16
NW = NC * NS
BPW = B // NW
CHUNK = 128
NCHUNK = BPW // CHUNK
NGROUP = BPW // 16


def _body(item_idx_hbm, cat_idx_hbm, brand_idx_hbm,
          item_tbl_hbm, cat_flat_hbm, brand_flat_hbm,
          out_hbm,
          ii_v, ic_v, ib_v, ig, tb, cat_vm, brand_vm, sem, gsem, osem):
    wid = lax.axis_index("s") * NC + lax.axis_index("c")
    base = wid * NCHUNK

    stage = [
        pltpu.make_async_copy(item_idx_hbm.at[pl.ds(base, NCHUNK)], ii_v, sem),
        pltpu.make_async_copy(cat_idx_hbm.at[pl.ds(base, NCHUNK)], ic_v, sem),
        pltpu.make_async_copy(brand_idx_hbm.at[pl.ds(base, NCHUNK)], ib_v, sem),
        pltpu.make_async_copy(cat_flat_hbm, cat_vm, sem),
        pltpu.make_async_copy(brand_flat_hbm, brand_vm, sem),
    ]
    for c in stage:
        c.start()
    for c in stage:
        c.wait()

    for j in range(NCHUNK):
        for i in range(CHUNK // 16):
            sl = pl.ds(i * 16, 16)
            ii_v[j, sl] = ii_v[j, sl] + 1

    copies = [
        pltpu.make_async_copy(
            item_tbl_hbm.at[ii_v.at[j]],
            ig.at[pl.ds(j * CHUNK, CHUNK)], gsem)
        for j in range(NCHUNK)
    ]
    for c in copies:
        c.start()

    iota16 = lax.iota(jnp.int32, 16)
    out_cols = [jnp.full((16,), c, jnp.int32) for c in range(TAIL_D)]
    for g in range(NGROUP):
        j, i = divmod(g, CHUNK // 16)
        sl = pl.ds(i * 16, 16)
        rows16 = iota16 + (g * 16)
        ic6 = ic_v[j, sl] * CAT_D
        ib10 = ib_v[j, sl] * BRAND_D
        for c in range(TAIL_D):
            if c < CAT_D:
                v = plsc.load_gather(cat_vm, [ic6 + (CAT_D + c)])
            else:
                v = plsc.load_gather(brand_vm, [ib10 + (BRAND_D + c - CAT_D)])
            plsc.store_scatter(tb, [rows16, out_cols[c]], v)

    rows = pl.ds(wid * BPW, BPW)
    tail_out = pltpu.make_async_copy(
        tb, out_hbm.at[rows, pl.ds(ITEM_D, TAIL_D)], osem)
    tail_out.start()

    for c in copies:
        c.wait()
    item_out = pltpu.make_async_copy(
        ig, out_hbm.at[rows, pl.ds(0, ITEM_D)], osem)
    item_out.start()

    tail_out.wait()
    item_out.wait()


def kernel(item_id, category_id, brand_id, item_table, cat_table, brand_table):
    mesh = plsc.VectorSubcoreMesh(core_axis_name="c", subcore_axis_name="s")
    k = functools.partial(
        pl.kernel,
        mesh=mesh,
        compiler_params=pltpu.CompilerParams(use_tc_tiling_on_sc=False,
                                             needs_layout_passes=False,
                                             disable_bounds_checks=True,
                                             skip_device_barrier=True),
        out_type=jax.ShapeDtypeStruct((B, OUT_D), jnp.float32),
        scratch_types=[
            pltpu.VMEM((NCHUNK, CHUNK), jnp.int32),
            pltpu.VMEM((NCHUNK, CHUNK), jnp.int32),
            pltpu.VMEM((NCHUNK, CHUNK), jnp.int32),
            pltpu.VMEM((BPW, ITEM_D), jnp.float32),
            pltpu.VMEM((BPW, TAIL_D), jnp.float32),
            pltpu.VMEM((CAT_WORDS,), jnp.float32),
            pltpu.VMEM((BRAND_WORDS,), jnp.float32),
            pltpu.SemaphoreType.DMA,
            pltpu.SemaphoreType.DMA,
            pltpu.SemaphoreType.DMA,
        ],
    )(_body)
    return k(item_id.reshape(B // CHUNK, CHUNK),
             category_id.reshape(B // CHUNK, CHUNK),
             brand_id.reshape(B // CHUNK, CHUNK),
             item_table,
             cat_table.reshape(-1),
             brand_table.reshape(-1))


# speedup vs baseline: 1.3234x; 1.2149x over previous
import functools

import jax
import jax.numpy as jnp
from jax import lax
from jax.experimental import pallas as pl
from jax.experimental.pallas import tpu as pltpu
from jax.experimental.pallas import tpu_sc as plsc

B = 16384
ITEM_D = 32
CAT_D = 6
BRAND_D = 10
TAIL_D = CAT_D + BRAND_D
OUT_D = ITEM_D + TAIL_D
PAD_D = 128

CAT_WORDS = 101 * CAT_D
BRAND_OFF = 608
BRAND_WORDS = 201 * BRAND_D
TBL_WORDS = BRAND_OFF + BRAND_WORDS

NC = 2
NS = 16
NW = NC * NS
BPW = B // NW
CHUNK = 128
NCHUNK = BPW // CHUNK
NGROUP = BPW // 16


def _body(item_idx_hbm, cat_idx_hbm, brand_idx_hbm,
          item_tbl_hbm, cat_flat_hbm, brand_flat_hbm,
          out_hbm,
          idx_v, ig, tb, tbl_vm, sem, gsem, osem):
    wid = lax.axis_index("s") * NC + lax.axis_index("c")
    base = wid * BPW

    stage = [
        pltpu.make_async_copy(item_idx_hbm.at[pl.ds(base, BPW)],
                              idx_v.at[pl.ds(0, BPW)], sem),
        pltpu.make_async_copy(cat_idx_hbm.at[pl.ds(base, BPW)],
                              idx_v.at[pl.ds(BPW, BPW)], sem),
        pltpu.make_async_copy(brand_idx_hbm.at[pl.ds(base, BPW)],
                              idx_v.at[pl.ds(2 * BPW, BPW)], sem),
        pltpu.make_async_copy(cat_flat_hbm,
                              tbl_vm.at[pl.ds(0, CAT_WORDS)], sem),
        pltpu.make_async_copy(brand_flat_hbm,
                              tbl_vm.at[pl.ds(BRAND_OFF, BRAND_WORDS)], sem),
    ]
    for c in stage:
        c.start()
    for c in stage:
        c.wait()

    for i in range(BPW // 16):
        sl = pl.ds(i * 16, 16)
        idx_v[sl] = idx_v[sl] + 1

    copies = [
        pltpu.make_async_copy(
            item_tbl_hbm.at[idx_v.at[pl.ds(j * CHUNK, CHUNK)]],
            ig.at[pl.ds(j * CHUNK, CHUNK)], gsem)
        for j in range(NCHUNK)
    ]
    for c in copies:
        c.start()

    iota16 = lax.iota(jnp.int32, 16)
    out_cols = [jnp.full((16,), c, jnp.int32) for c in range(TAIL_D)]
    for g in range(NGROUP):
        sl = pl.ds(g * 16, 16)
        rows16 = iota16 + (g * 16)
        ic6 = idx_v[pl.ds(BPW + g * 16, 16)] * CAT_D
        ib10 = idx_v[pl.ds(2 * BPW + g * 16, 16)] * BRAND_D
        for c in range(TAIL_D):
            if c < CAT_D:
                v = plsc.load_gather(tbl_vm, [ic6 + (CAT_D + c)])
            else:
                v = plsc.load_gather(
                    tbl_vm, [ib10 + (BRAND_OFF + BRAND_D + c - CAT_D)])
            plsc.store_scatter(tb, [rows16, out_cols[c]], v)

    rows = pl.ds(wid * BPW, BPW)
    tail_out = pltpu.make_async_copy(
        tb, out_hbm.at[rows, pl.ds(ITEM_D, TAIL_D)], osem)
    tail_out.start()

    for c in copies:
        c.wait()
    item_out = pltpu.make_async_copy(
        ig, out_hbm.at[rows, pl.ds(0, ITEM_D)], osem)
    item_out.start()

    tail_out.wait()
    item_out.wait()


def kernel(item_id, category_id, brand_id, item_table, cat_table, brand_table):
    mesh = plsc.VectorSubcoreMesh(core_axis_name="c", subcore_axis_name="s")
    k = functools.partial(
        pl.kernel,
        mesh=mesh,
        compiler_params=pltpu.CompilerParams(use_tc_tiling_on_sc=False,
                                             needs_layout_passes=False,
                                             disable_bounds_checks=True,
                                             disable_semaphore_checks=True,
                                             skip_device_barrier=True),
        out_type=jax.ShapeDtypeStruct((B, PAD_D), jnp.float32),
        scratch_types=[
            pltpu.VMEM((3 * BPW,), jnp.int32),
            pltpu.VMEM((BPW, ITEM_D), jnp.float32),
            pltpu.VMEM((BPW, TAIL_D), jnp.float32),
            pltpu.VMEM((TBL_WORDS,), jnp.float32),
            pltpu.SemaphoreType.DMA,
            pltpu.SemaphoreType.DMA,
            pltpu.SemaphoreType.DMA,
        ],
    )(_body)
    out = k(item_id, category_id, brand_id,
            item_table,
            cat_table.reshape(-1),
            brand_table.reshape(-1))
    return out[:, :OUT_D]
